# baseline (speedup 1.0000x reference)
import functools
import jax
import jax.numpy as jnp
from jax.experimental import pallas as pl
from jax.experimental.pallas import tpu as pltpu


def _argmin_parts(dist, iota_f):
    m = jnp.min(dist, axis=1, keepdims=True)
    cand = jnp.where(dist == m, iota_f, jnp.float32(dist.shape[1]))
    idx = jnp.min(cand, axis=1, keepdims=True)
    enc = (iota_f == idx).astype(jnp.float32)
    return m, enc


def _dist(v, emb):
    xs = jnp.sum(v * v, axis=1, keepdims=True)
    cross = jax.lax.dot_general(v, emb, (((1,), (1,)), ((), ())),
                                preferred_element_type=jnp.float32)
    es = jnp.sum(emb * emb, axis=1)[None, :]
    return xs - 2.0 * cross + es


def _body(f_ref, w_ref, b_ref, embt_ref, embp_ref,
          tout_ref, pout_ref, loss_ref, *, loss_scale, d):
    i = pl.program_id(0)
    x = f_ref[...]
    TP = jnp.dot(x, w_ref[...], preferred_element_type=jnp.float32) + b_ref[...]
    T = TP[:, :d]
    P = TP[:, d:]

    embT = embt_ref[...]
    embP = embp_ref[...]

    distT = _dist(T, embT)
    iota_f = jax.lax.broadcasted_iota(jnp.int32, distT.shape, 1).astype(jnp.float32)
    mT, encT = _argmin_parts(distT, iota_f)

    neg = -distT
    mx = jnp.max(neg, axis=1, keepdims=True)
    lse = mx + jnp.log(jnp.sum(jnp.exp(neg - mx), axis=1, keepdims=True))
    w = neg - lse

    tout_ref[...] = jnp.dot(0.5 * (w + encT), embT,
                            preferred_element_type=jnp.float32)

    distP = _dist(P, embP)
    mP, encP = _argmin_parts(distP, iota_f)
    pout_ref[...] = jnp.dot(encP, embP, preferred_element_type=jnp.float32)

    partial = ((jnp.sum(mT) + jnp.sum(mP)) * loss_scale).reshape(1, 1)

    @pl.when(i == 0)
    def _():
        loss_ref[...] = partial

    @pl.when(i != 0)
    def _():
        loss_ref[...] = loss_ref[...] + partial


def kernel(f, W_T, b_T, W_P, b_P, emb_T, emb_P):
    B, L, E = f.shape
    N = B * L
    D = W_T.shape[1]
    BT = 1024
    ff = f.reshape(N, E)
    W = jnp.concatenate([W_T, W_P], axis=1)
    b = jnp.concatenate([b_T, b_P]).reshape(1, 2 * D)
    loss_scale = 1.25 / (N * D)

    grid = (N // BT,)
    const_spec = lambda shape: pl.BlockSpec(shape, lambda i: (0, 0))
    T_out, P_out, loss = pl.pallas_call(
        functools.partial(_body, loss_scale=loss_scale, d=D),
        grid=grid,
        in_specs=[
            pl.BlockSpec((BT, E), lambda i: (i, 0)),
            const_spec((E, 2 * D)),
            const_spec((1, 2 * D)),
            const_spec(emb_T.shape),
            const_spec(emb_P.shape),
        ],
        out_specs=[
            pl.BlockSpec((BT, D), lambda i: (i, 0)),
            pl.BlockSpec((BT, D), lambda i: (i, 0)),
            pl.BlockSpec((1, 1), lambda i: (0, 0)),
        ],
        out_shape=[
            jax.ShapeDtypeStruct((N, D), jnp.float32),
            jax.ShapeDtypeStruct((N, D), jnp.float32),
            jax.ShapeDtypeStruct((1, 1), jnp.float32),
        ],
    )(ff, W, b, emb_T, emb_P)

    return T_out.reshape(B, L, D), P_out.reshape(B, L, D), loss[0, 0]


# speedup vs baseline: 1.1257x; 1.1257x over previous
import functools
import jax
import jax.numpy as jnp
from jax.experimental import pallas as pl
from jax.experimental.pallas import tpu as pltpu


def _argmin_parts(dist, iota_f):
    m = jnp.min(dist, axis=1, keepdims=True)
    cand = jnp.where(dist == m, iota_f, jnp.float32(dist.shape[1]))
    idx = jnp.min(cand, axis=1, keepdims=True)
    enc = (iota_f == idx).astype(jnp.float32)
    return m, enc


def _dist(v, emb):
    xs = jnp.sum(v * v, axis=1, keepdims=True)
    cross = jax.lax.dot_general(v, emb, (((1,), (1,)), ((), ())),
                                preferred_element_type=jnp.float32)
    es = jnp.sum(emb * emb, axis=1)[None, :]
    return xs - 2.0 * cross + es


def _body(f_ref, w_ref, b_ref, embt_ref, embp_ref,
          tout_ref, pout_ref, loss_ref, *, loss_scale, d):
    i = pl.program_id(0)
    x = f_ref[...]
    TP = jnp.dot(x, w_ref[...], preferred_element_type=jnp.float32) + b_ref[...]
    T = TP[:, :d]
    P = TP[:, d:]

    embT = embt_ref[...]
    embP = embp_ref[...]

    distT = _dist(T, embT)
    iota_f = jax.lax.broadcasted_iota(jnp.int32, distT.shape, 1).astype(jnp.float32)
    mT, encT = _argmin_parts(distT, iota_f)

    neg = -distT
    mx = jnp.max(neg, axis=1, keepdims=True)
    lse = mx + jnp.log(jnp.sum(jnp.exp(neg - mx), axis=1, keepdims=True))
    w = neg - lse

    tout_ref[...] = jnp.dot(0.5 * (w + encT), embT,
                            preferred_element_type=jnp.float32)

    distP = _dist(P, embP)
    mP, encP = _argmin_parts(distP, iota_f)
    pout_ref[...] = jnp.dot(encP, embP, preferred_element_type=jnp.float32)

    partial = ((jnp.sum(mT) + jnp.sum(mP)) * loss_scale).reshape(1, 1)

    @pl.when(i == 0)
    def _():
        loss_ref[...] = partial

    @pl.when(i != 0)
    def _():
        loss_ref[...] = loss_ref[...] + partial


def kernel(f, W_T, b_T, W_P, b_P, emb_T, emb_P):
    B, L, E = f.shape
    N = B * L
    D = W_T.shape[1]
    BT = 4096
    ff = f.reshape(N, E)
    W = jnp.concatenate([W_T, W_P], axis=1)
    b = jnp.concatenate([b_T, b_P]).reshape(1, 2 * D)
    loss_scale = 1.25 / (N * D)

    grid = (N // BT,)
    const_spec = lambda shape: pl.BlockSpec(shape, lambda i: (0, 0))
    T_out, P_out, loss = pl.pallas_call(
        functools.partial(_body, loss_scale=loss_scale, d=D),
        grid=grid,
        in_specs=[
            pl.BlockSpec((BT, E), lambda i: (i, 0)),
            const_spec((E, 2 * D)),
            const_spec((1, 2 * D)),
            const_spec(emb_T.shape),
            const_spec(emb_P.shape),
        ],
        out_specs=[
            pl.BlockSpec((BT, D), lambda i: (i, 0)),
            pl.BlockSpec((BT, D), lambda i: (i, 0)),
            pl.BlockSpec((1, 1), lambda i: (0, 0)),
        ],
        out_shape=[
            jax.ShapeDtypeStruct((N, D), jnp.float32),
            jax.ShapeDtypeStruct((N, D), jnp.float32),
            jax.ShapeDtypeStruct((1, 1), jnp.float32),
        ],
    )(ff, W, b, emb_T, emb_P)

    return T_out.reshape(B, L, D), P_out.reshape(B, L, D), loss[0, 0]


# speedup vs baseline: 1.1737x; 1.0427x over previous
import functools
import jax
import jax.numpy as jnp
from jax.experimental import pallas as pl
from jax.experimental.pallas import tpu as pltpu


def _argmin_parts(g, iota_f):
    m = jnp.min(g, axis=1, keepdims=True)
    cand = jnp.where(g == m, iota_f, jnp.float32(g.shape[1]))
    idx = jnp.min(cand, axis=1, keepdims=True)
    enc = (iota_f == idx).astype(jnp.float32)
    return m, enc


def _body(f_ref, w_ref, b_ref, ebd_ref, embt_ref, embp_ref,
          tout_ref, pout_ref, loss_ref, *, loss_scale, d, k):
    i = pl.program_id(0)
    x = f_ref[...]
    TP = jnp.dot(x, w_ref[...], preferred_element_type=jnp.float32) + b_ref[...]

    ebd = ebd_ref[...]
    embT = embt_ref[...]
    embP = embp_ref[...]
    es = jnp.concatenate(
        [jnp.sum(embT * embT, axis=1), jnp.sum(embP * embP, axis=1)])[None, :]
    cross = jnp.dot(TP, ebd, preferred_element_type=jnp.float32)
    g = es - 2.0 * cross
    gT = g[:, :k]
    gP = g[:, k:]

    iota_f = jax.lax.broadcasted_iota(jnp.int32, gT.shape, 1).astype(jnp.float32)
    mT, encT = _argmin_parts(gT, iota_f)
    mP, encP = _argmin_parts(gP, iota_f)

    e = jnp.exp(mT - gT)
    lse = jnp.log(jnp.sum(e, axis=1, keepdims=True))
    w = (mT - gT) - lse

    tout_ref[...] = jnp.dot(0.5 * (w + encT), embT,
                            preferred_element_type=jnp.float32)
    pout_ref[...] = jnp.dot(encP, embP, preferred_element_type=jnp.float32)

    sumsq = jnp.sum(TP * TP)
    partial = ((sumsq + jnp.sum(mT) + jnp.sum(mP)) * loss_scale).reshape(1, 1)

    @pl.when(i == 0)
    def _():
        loss_ref[...] = partial

    @pl.when(i != 0)
    def _():
        loss_ref[...] = loss_ref[...] + partial


def kernel(f, W_T, b_T, W_P, b_P, emb_T, emb_P):
    B, L, E = f.shape
    N = B * L
    D = W_T.shape[1]
    K = emb_T.shape[0]
    BT = 4096
    ff = f.reshape(N, E)
    W = jnp.concatenate([W_T, W_P], axis=1)
    b = jnp.concatenate([b_T, b_P]).reshape(1, 2 * D)
    z = jnp.zeros((D, K), jnp.float32)
    ebd = jnp.concatenate(
        [jnp.concatenate([emb_T.T, z], axis=1),
         jnp.concatenate([z, emb_P.T], axis=1)], axis=0)
    loss_scale = 1.25 / (N * D)

    grid = (N // BT,)
    const_spec = lambda shape: pl.BlockSpec(shape, lambda i: (0, 0))
    T_out, P_out, loss = pl.pallas_call(
        functools.partial(_body, loss_scale=loss_scale, d=D, k=K),
        grid=grid,
        in_specs=[
            pl.BlockSpec((BT, E), lambda i: (i, 0)),
            const_spec((E, 2 * D)),
            const_spec((1, 2 * D)),
            const_spec((2 * D, 2 * K)),
            const_spec(emb_T.shape),
            const_spec(emb_P.shape),
        ],
        out_specs=[
            pl.BlockSpec((BT, D), lambda i: (i, 0)),
            pl.BlockSpec((BT, D), lambda i: (i, 0)),
            pl.BlockSpec((1, 1), lambda i: (0, 0)),
        ],
        out_shape=[
            jax.ShapeDtypeStruct((N, D), jnp.float32),
            jax.ShapeDtypeStruct((N, D), jnp.float32),
            jax.ShapeDtypeStruct((1, 1), jnp.float32),
        ],
    )(ff, W, b, ebd, emb_T, emb_P)

    return T_out.reshape(B, L, D), P_out.reshape(B, L, D), loss[0, 0]
